# baseline (speedup 1.0000x reference)
import functools

import jax
import jax.numpy as jnp
from jax.experimental import pallas as pl
from jax.experimental.pallas import tpu as pltpu

T = 2048
H = 1024
NH = 16
HD = H // NH
QC = 512
KVC = 256
E = 8
TOPK = 2
FF = 512
AM = 0.125
RM = 0.22
EPS = 1e-06

BT = 512
BQ = 512
BK = 512
NQB = T // BQ
NKB = T // BK


def _rms(x, w):
    var = jnp.mean(x * x, axis=-1, keepdims=True)
    return x * jax.lax.rsqrt(var + EPS) * w


def _pre_attn_kernel(x_ref, ln1_ref, wd_ref, wq_ref, wk_ref, wv_ref,
                     q_ref, k_ref, v_ref):
    h = _rms(x_ref[...], ln1_ref[...])
    d = jnp.dot(h, wd_ref[...], preferred_element_type=jnp.float32)
    q_ref[...] = jnp.dot(d[:, :QC], wq_ref[...],
                         preferred_element_type=jnp.float32)
    k_ref[...] = jnp.dot(d[:, QC:QC + KVC], wk_ref[...],
                         preferred_element_type=jnp.float32)
    v_ref[...] = jnp.dot(d[:, QC + KVC:], wv_ref[...],
                         preferred_element_type=jnp.float32)


def _flash_attn_kernel(q_ref, k_ref, v_ref, o_ref, m_ref, l_ref, acc_ref):
    i = pl.program_id(1)
    j = pl.program_id(2)

    @pl.when(j == 0)
    def _init():
        m_ref[...] = jnp.full_like(m_ref, -1e30)
        l_ref[...] = jnp.zeros_like(l_ref)
        acc_ref[...] = jnp.zeros_like(acc_ref)

    @pl.when(j <= i)
    def _body():
        q = q_ref[0]
        k = k_ref[0]
        s = jax.lax.dot_general(q, k, (((1,), (1,)), ((), ())),
                                preferred_element_type=jnp.float32) * AM
        row = i * BQ + jax.lax.broadcasted_iota(jnp.int32, (BQ, BK), 0)
        col = j * BK + jax.lax.broadcasted_iota(jnp.int32, (BQ, BK), 1)
        s = jnp.where(col <= row, s, -1e30)
        m_prev = m_ref[:, :1]
        m_new = jnp.maximum(m_prev, jnp.max(s, axis=1, keepdims=True))
        p = jnp.exp(s - m_new)
        alpha = jnp.exp(m_prev - m_new)
        l_new = alpha * l_ref[:, :1] + jnp.sum(p, axis=1, keepdims=True)
        acc_ref[...] = acc_ref[...] * alpha + jnp.dot(
            p, v_ref[0], preferred_element_type=jnp.float32)
        m_ref[...] = jnp.broadcast_to(m_new, m_ref.shape)
        l_ref[...] = jnp.broadcast_to(l_new, l_ref.shape)

    @pl.when(j == NKB - 1)
    def _final():
        o_ref[0] = acc_ref[...] / l_ref[:, :1]


def _post_attn_kernel(attn_ref, res_ref, ln2_ref, wo_ref, rw_ref,
                      hid_ref, h2_ref, we_ref):
    o = jnp.dot(attn_ref[...], wo_ref[...], preferred_element_type=jnp.float32)
    hidden = res_ref[...] + o * RM
    hid_ref[...] = hidden
    h2 = _rms(hidden, ln2_ref[...])
    h2_ref[...] = h2
    logits = jnp.dot(h2, rw_ref[...], preferred_element_type=jnp.float32)
    iota = jax.lax.broadcasted_iota(jnp.int32, logits.shape, 1)
    m1 = jnp.max(logits, axis=1, keepdims=True)
    i1 = jnp.min(jnp.where(logits == m1, iota, E), axis=1, keepdims=True)
    masked = jnp.where(iota == i1, -1e30, logits)
    m2 = jnp.max(masked, axis=1, keepdims=True)
    i2 = jnp.min(jnp.where(masked == m2, iota, E), axis=1, keepdims=True)
    e2 = jnp.exp(m2 - m1)
    rw1 = 1.0 / (1.0 + e2)
    rw2 = e2 / (1.0 + e2)
    we_ref[...] = (jnp.where(iota == i1, rw1, 0.0)
                   + jnp.where(iota == i2, rw2, 0.0))


def _moe_kernel(h2_ref, res2_ref, we_ref, w1_ref, w2_ref, out_ref):
    e = pl.program_id(1)

    @pl.when(e == 0)
    def _init():
        out_ref[...] = res2_ref[...]

    x1 = jnp.dot(h2_ref[...], w1_ref[0], preferred_element_type=jnp.float32)
    gate = x1[:, :FF]
    up = x1[:, FF:]
    act = gate * jax.lax.logistic(gate) * up
    eout = jnp.dot(act, w2_ref[0], preferred_element_type=jnp.float32)
    iota = jax.lax.broadcasted_iota(jnp.int32, we_ref.shape, 1)
    w_col = jnp.sum(jnp.where(iota == e, we_ref[...], 0.0),
                    axis=1, keepdims=True)
    out_ref[...] += w_col * eout * RM


def kernel(positions, hidden_states, residual, ln1_w, ln2_w, w_down, w_q_up,
           w_k_up, w_v_up, w_o, router_w, w1, w2):
    del positions, residual
    f32 = jnp.float32
    ln1 = ln1_w.reshape(1, H)
    ln2 = ln2_w.reshape(1, H)
    wdT = w_down.T
    wqT = w_q_up.T
    wkT = w_k_up.T
    wvT = w_v_up.T
    woT = w_o.T
    rwT = router_w.T
    w1T = w1.transpose(0, 2, 1)
    w2T = w2.transpose(0, 2, 1)

    nbt = T // BT
    q, k, v = pl.pallas_call(
        _pre_attn_kernel,
        grid=(nbt,),
        in_specs=[
            pl.BlockSpec((BT, H), lambda i: (i, 0)),
            pl.BlockSpec((1, H), lambda i: (0, 0)),
            pl.BlockSpec((H, QC + 2 * KVC), lambda i: (0, 0)),
            pl.BlockSpec((QC, H), lambda i: (0, 0)),
            pl.BlockSpec((KVC, H), lambda i: (0, 0)),
            pl.BlockSpec((KVC, H), lambda i: (0, 0)),
        ],
        out_specs=[
            pl.BlockSpec((BT, H), lambda i: (i, 0)),
            pl.BlockSpec((BT, H), lambda i: (i, 0)),
            pl.BlockSpec((BT, H), lambda i: (i, 0)),
        ],
        out_shape=[jax.ShapeDtypeStruct((T, H), f32)] * 3,
    )(hidden_states, ln1, wdT, wqT, wkT, wvT)

    qh = q.reshape(T, NH, HD).transpose(1, 0, 2)
    kh = k.reshape(T, NH, HD).transpose(1, 0, 2)
    vh = v.reshape(T, NH, HD).transpose(1, 0, 2)

    attn = pl.pallas_call(
        _flash_attn_kernel,
        grid=(NH, NQB, NKB),
        in_specs=[
            pl.BlockSpec((1, BQ, HD), lambda h, i, j: (h, i, 0)),
            pl.BlockSpec((1, BK, HD), lambda h, i, j: (h, j, 0)),
            pl.BlockSpec((1, BK, HD), lambda h, i, j: (h, j, 0)),
        ],
        out_specs=pl.BlockSpec((1, BQ, HD), lambda h, i, j: (h, i, 0)),
        out_shape=jax.ShapeDtypeStruct((NH, T, HD), f32),
        scratch_shapes=[
            pltpu.VMEM((BQ, 128), f32),
            pltpu.VMEM((BQ, 128), f32),
            pltpu.VMEM((BQ, HD), f32),
        ],
    )(qh, kh, vh)

    attn2d = attn.transpose(1, 0, 2).reshape(T, H)

    res2, h2, we = pl.pallas_call(
        _post_attn_kernel,
        grid=(nbt,),
        in_specs=[
            pl.BlockSpec((BT, H), lambda i: (i, 0)),
            pl.BlockSpec((BT, H), lambda i: (i, 0)),
            pl.BlockSpec((1, H), lambda i: (0, 0)),
            pl.BlockSpec((H, H), lambda i: (0, 0)),
            pl.BlockSpec((H, E), lambda i: (0, 0)),
        ],
        out_specs=[
            pl.BlockSpec((BT, H), lambda i: (i, 0)),
            pl.BlockSpec((BT, H), lambda i: (i, 0)),
            pl.BlockSpec((BT, E), lambda i: (i, 0)),
        ],
        out_shape=[
            jax.ShapeDtypeStruct((T, H), f32),
            jax.ShapeDtypeStruct((T, H), f32),
            jax.ShapeDtypeStruct((T, E), f32),
        ],
    )(attn2d, hidden_states, ln2, woT, rwT)

    out = pl.pallas_call(
        _moe_kernel,
        grid=(nbt, E),
        in_specs=[
            pl.BlockSpec((BT, H), lambda i, e: (i, 0)),
            pl.BlockSpec((BT, H), lambda i, e: (i, 0)),
            pl.BlockSpec((BT, E), lambda i, e: (i, 0)),
            pl.BlockSpec((1, H, 2 * FF), lambda i, e: (e, 0, 0)),
            pl.BlockSpec((1, FF, H), lambda i, e: (e, 0, 0)),
        ],
        out_specs=pl.BlockSpec((BT, H), lambda i, e: (i, 0)),
        out_shape=jax.ShapeDtypeStruct((T, H), f32),
    )(h2, res2, we, w1T, w2T)

    return (out, res2)


# speedup vs baseline: 1.6424x; 1.6424x over previous
import functools

import jax
import jax.numpy as jnp
from jax.experimental import pallas as pl
from jax.experimental.pallas import tpu as pltpu

T = 2048
H = 1024
NH = 16
HD = H // NH
QC = 512
KVC = 256
E = 8
TOPK = 2
FF = 512
AM = 0.125
RM = 0.22
EPS = 1e-06

BT = 512
BQ = 512
BK = 512
NQB = T // BQ
NKB = T // BK


def _rms(x, w):
    var = jnp.mean(x * x, axis=-1, keepdims=True)
    return x * jax.lax.rsqrt(var + EPS) * w


def _pre_attn_kernel(x_ref, ln1_ref, wd_ref, wq_ref, wk_ref, wv_ref,
                     q_ref, k_ref, v_ref):
    h = _rms(x_ref[...], ln1_ref[...])
    d = jnp.dot(h, wd_ref[...], preferred_element_type=jnp.float32)
    q_ref[...] = jnp.dot(d[:, :QC], wq_ref[...],
                         preferred_element_type=jnp.float32)
    k_ref[...] = jnp.dot(d[:, QC:QC + KVC], wk_ref[...],
                         preferred_element_type=jnp.float32)
    v_ref[...] = jnp.dot(d[:, QC + KVC:], wv_ref[...],
                         preferred_element_type=jnp.float32)


def _flash_attn_kernel(q_ref, k_ref, v_ref, o_ref, m_ref, l_ref, acc_ref):
    i = pl.program_id(0)
    j = pl.program_id(1)

    @pl.when(j == 0)
    def _init():
        m_ref[...] = jnp.full_like(m_ref, -1e30)
        l_ref[...] = jnp.zeros_like(l_ref)
        acc_ref[...] = jnp.zeros_like(acc_ref)

    @pl.when(j <= i)
    def _body():
        diag = j == i
        row = i * BQ + jax.lax.broadcasted_iota(jnp.int32, (BQ, BK), 0)
        col = j * BK + jax.lax.broadcasted_iota(jnp.int32, (BQ, BK), 1)
        keep = jnp.logical_or(jnp.logical_not(diag), col <= row)
        for h in range(NH):
            sl = slice(h * HD, (h + 1) * HD)
            qh = q_ref[:, sl]
            kh = k_ref[:, sl]
            s = jax.lax.dot_general(qh, kh, (((1,), (1,)), ((), ())),
                                    preferred_element_type=jnp.float32) * AM
            s = jnp.where(keep, s, -1e30)
            m_prev = m_ref[:, h:h + 1]
            m_new = jnp.maximum(m_prev, jnp.max(s, axis=1, keepdims=True))
            p = jnp.exp(s - m_new)
            alpha = jnp.exp(m_prev - m_new)
            l_ref[:, h:h + 1] = (alpha * l_ref[:, h:h + 1]
                                 + jnp.sum(p, axis=1, keepdims=True))
            acc_ref[:, sl] = acc_ref[:, sl] * alpha + jnp.dot(
                p, v_ref[:, sl], preferred_element_type=jnp.float32)
            m_ref[:, h:h + 1] = m_new

    @pl.when(j == NKB - 1)
    def _final():
        for h in range(NH):
            sl = slice(h * HD, (h + 1) * HD)
            o_ref[:, sl] = acc_ref[:, sl] / l_ref[:, h:h + 1]


def _post_attn_kernel(attn_ref, res_ref, ln2_ref, wo_ref, rw_ref,
                      hid_ref, h2_ref, we_ref):
    o = jnp.dot(attn_ref[...], wo_ref[...], preferred_element_type=jnp.float32)
    hidden = res_ref[...] + o * RM
    hid_ref[...] = hidden
    h2 = _rms(hidden, ln2_ref[...])
    h2_ref[...] = h2
    logits = jnp.dot(h2, rw_ref[...], preferred_element_type=jnp.float32)
    iota = jax.lax.broadcasted_iota(jnp.int32, logits.shape, 1)
    m1 = jnp.max(logits, axis=1, keepdims=True)
    i1 = jnp.min(jnp.where(logits == m1, iota, E), axis=1, keepdims=True)
    masked = jnp.where(iota == i1, -1e30, logits)
    m2 = jnp.max(masked, axis=1, keepdims=True)
    i2 = jnp.min(jnp.where(masked == m2, iota, E), axis=1, keepdims=True)
    e2 = jnp.exp(m2 - m1)
    rw1 = 1.0 / (1.0 + e2)
    rw2 = e2 / (1.0 + e2)
    we_ref[...] = (jnp.where(iota == i1, rw1, 0.0)
                   + jnp.where(iota == i2, rw2, 0.0))


def _moe_kernel(h2_ref, res2_ref, we_ref, w1_ref, w2_ref, out_ref):
    e = pl.program_id(1)

    @pl.when(e == 0)
    def _init():
        out_ref[...] = res2_ref[...]

    x1 = jnp.dot(h2_ref[...], w1_ref[0], preferred_element_type=jnp.float32)
    gate = x1[:, :FF]
    up = x1[:, FF:]
    act = gate * jax.lax.logistic(gate) * up
    eout = jnp.dot(act, w2_ref[0], preferred_element_type=jnp.float32)
    iota = jax.lax.broadcasted_iota(jnp.int32, we_ref.shape, 1)
    w_col = jnp.sum(jnp.where(iota == e, we_ref[...], 0.0),
                    axis=1, keepdims=True)
    out_ref[...] += w_col * eout * RM


def kernel(positions, hidden_states, residual, ln1_w, ln2_w, w_down, w_q_up,
           w_k_up, w_v_up, w_o, router_w, w1, w2):
    del positions, residual
    f32 = jnp.float32
    ln1 = ln1_w.reshape(1, H)
    ln2 = ln2_w.reshape(1, H)
    wdT = w_down.T
    wqT = w_q_up.T
    wkT = w_k_up.T
    wvT = w_v_up.T
    woT = w_o.T
    rwT = router_w.T
    w1T = w1.transpose(0, 2, 1)
    w2T = w2.transpose(0, 2, 1)

    nbt = T // BT
    q, k, v = pl.pallas_call(
        _pre_attn_kernel,
        grid=(nbt,),
        in_specs=[
            pl.BlockSpec((BT, H), lambda i: (i, 0)),
            pl.BlockSpec((1, H), lambda i: (0, 0)),
            pl.BlockSpec((H, QC + 2 * KVC), lambda i: (0, 0)),
            pl.BlockSpec((QC, H), lambda i: (0, 0)),
            pl.BlockSpec((KVC, H), lambda i: (0, 0)),
            pl.BlockSpec((KVC, H), lambda i: (0, 0)),
        ],
        out_specs=[
            pl.BlockSpec((BT, H), lambda i: (i, 0)),
            pl.BlockSpec((BT, H), lambda i: (i, 0)),
            pl.BlockSpec((BT, H), lambda i: (i, 0)),
        ],
        out_shape=[jax.ShapeDtypeStruct((T, H), f32)] * 3,
    )(hidden_states, ln1, wdT, wqT, wkT, wvT)

    attn2d = pl.pallas_call(
        _flash_attn_kernel,
        grid=(NQB, NKB),
        in_specs=[
            pl.BlockSpec((BQ, H), lambda i, j: (i, 0)),
            pl.BlockSpec((BK, H), lambda i, j: (j, 0)),
            pl.BlockSpec((BK, H), lambda i, j: (j, 0)),
        ],
        out_specs=pl.BlockSpec((BQ, H), lambda i, j: (i, 0)),
        out_shape=jax.ShapeDtypeStruct((T, H), f32),
        scratch_shapes=[
            pltpu.VMEM((BQ, 128), f32),
            pltpu.VMEM((BQ, 128), f32),
            pltpu.VMEM((BQ, H), f32),
        ],
    )(q, k, v)

    res2, h2, we = pl.pallas_call(
        _post_attn_kernel,
        grid=(nbt,),
        in_specs=[
            pl.BlockSpec((BT, H), lambda i: (i, 0)),
            pl.BlockSpec((BT, H), lambda i: (i, 0)),
            pl.BlockSpec((1, H), lambda i: (0, 0)),
            pl.BlockSpec((H, H), lambda i: (0, 0)),
            pl.BlockSpec((H, E), lambda i: (0, 0)),
        ],
        out_specs=[
            pl.BlockSpec((BT, H), lambda i: (i, 0)),
            pl.BlockSpec((BT, H), lambda i: (i, 0)),
            pl.BlockSpec((BT, E), lambda i: (i, 0)),
        ],
        out_shape=[
            jax.ShapeDtypeStruct((T, H), f32),
            jax.ShapeDtypeStruct((T, H), f32),
            jax.ShapeDtypeStruct((T, E), f32),
        ],
    )(attn2d, hidden_states, ln2, woT, rwT)

    out = pl.pallas_call(
        _moe_kernel,
        grid=(nbt, E),
        in_specs=[
            pl.BlockSpec((BT, H), lambda i, e: (i, 0)),
            pl.BlockSpec((BT, H), lambda i, e: (i, 0)),
            pl.BlockSpec((BT, E), lambda i, e: (i, 0)),
            pl.BlockSpec((1, H, 2 * FF), lambda i, e: (e, 0, 0)),
            pl.BlockSpec((1, FF, H), lambda i, e: (e, 0, 0)),
        ],
        out_specs=pl.BlockSpec((BT, H), lambda i, e: (i, 0)),
        out_shape=jax.ShapeDtypeStruct((T, H), f32),
    )(h2, res2, we, w1T, w2T)

    return (out, res2)


# speedup vs baseline: 1.9410x; 1.1818x over previous
import functools

import jax
import jax.numpy as jnp
from jax.experimental import pallas as pl
from jax.experimental.pallas import tpu as pltpu

T = 2048
H = 1024
NH = 16
HD = H // NH
QC = 512
KVC = 256
E = 8
TOPK = 2
FF = 512
AM = 0.125
RM = 0.22
EPS = 1e-06

BT = 512
BQ = 512
BK = 512
NQB = T // BQ
NKB = T // BK


def _rms(x, w):
    var = jnp.mean(x * x, axis=-1, keepdims=True)
    return x * jax.lax.rsqrt(var + EPS) * w


def _dot_t(a, b):
    return jax.lax.dot_general(a, b, (((1,), (1,)), ((), ())),
                               preferred_element_type=jnp.float32)


def _pre_attn_kernel(x_ref, ln1_ref, wd_ref, wq_ref, wk_ref, wv_ref,
                     q_ref, k_ref, v_ref):
    h = _rms(x_ref[...], ln1_ref[...])
    d = _dot_t(h, wd_ref[...])
    q_ref[...] = _dot_t(d[:, :QC], wq_ref[...])
    k_ref[...] = _dot_t(d[:, QC:QC + KVC], wk_ref[...])
    v_ref[...] = _dot_t(d[:, QC + KVC:], wv_ref[...])


def _flash_attn_kernel(q_ref, k_ref, v_ref, o_ref, m_ref, l_ref, acc_ref):
    i = pl.program_id(0)
    j = pl.program_id(1)

    @pl.when(j == 0)
    def _init():
        m_ref[...] = jnp.full_like(m_ref, -1e30)
        l_ref[...] = jnp.zeros_like(l_ref)
        acc_ref[...] = jnp.zeros_like(acc_ref)

    @pl.when(j <= i)
    def _body():
        diag = j == i
        row = i * BQ + jax.lax.broadcasted_iota(jnp.int32, (BQ, BK), 0)
        col = j * BK + jax.lax.broadcasted_iota(jnp.int32, (BQ, BK), 1)
        keep = jnp.logical_or(jnp.logical_not(diag), col <= row)
        for h in range(NH):
            sl = slice(h * HD, (h + 1) * HD)
            qh = q_ref[:, sl]
            kh = k_ref[:, sl]
            s = jax.lax.dot_general(qh, kh, (((1,), (1,)), ((), ())),
                                    preferred_element_type=jnp.float32) * AM
            s = jnp.where(keep, s, -1e30)
            m_prev = m_ref[:, h:h + 1]
            m_new = jnp.maximum(m_prev, jnp.max(s, axis=1, keepdims=True))
            p = jnp.exp(s - m_new)
            alpha = jnp.exp(m_prev - m_new)
            l_ref[:, h:h + 1] = (alpha * l_ref[:, h:h + 1]
                                 + jnp.sum(p, axis=1, keepdims=True))
            acc_ref[:, sl] = acc_ref[:, sl] * alpha + jnp.dot(
                p, v_ref[:, sl], preferred_element_type=jnp.float32)
            m_ref[:, h:h + 1] = m_new

    @pl.when(j == NKB - 1)
    def _final():
        for h in range(NH):
            sl = slice(h * HD, (h + 1) * HD)
            o_ref[:, sl] = acc_ref[:, sl] / l_ref[:, h:h + 1]


def _post_attn_kernel(attn_ref, res_ref, ln2_ref, wo_ref, rw_ref,
                      hid_ref, h2_ref, we_ref):
    o = _dot_t(attn_ref[...], wo_ref[...])
    hidden = res_ref[...] + o * RM
    hid_ref[...] = hidden
    h2 = _rms(hidden, ln2_ref[...])
    h2_ref[...] = h2
    logits = _dot_t(h2, rw_ref[...])
    iota = jax.lax.broadcasted_iota(jnp.int32, logits.shape, 1)
    m1 = jnp.max(logits, axis=1, keepdims=True)
    i1 = jnp.min(jnp.where(logits == m1, iota, E), axis=1, keepdims=True)
    masked = jnp.where(iota == i1, -1e30, logits)
    m2 = jnp.max(masked, axis=1, keepdims=True)
    i2 = jnp.min(jnp.where(masked == m2, iota, E), axis=1, keepdims=True)
    e2 = jnp.exp(m2 - m1)
    rw1 = 1.0 / (1.0 + e2)
    rw2 = e2 / (1.0 + e2)
    we_ref[...] = (jnp.where(iota == i1, rw1, 0.0)
                   + jnp.where(iota == i2, rw2, 0.0))


def _moe_kernel(h2_ref, res2_ref, we_ref, w1_ref, w2_ref, out_ref):
    e = pl.program_id(1)

    @pl.when(e == 0)
    def _init():
        out_ref[...] = res2_ref[...]

    x1 = _dot_t(h2_ref[...], w1_ref[0])
    gate = x1[:, :FF]
    up = x1[:, FF:]
    act = gate * jax.lax.logistic(gate) * up
    eout = _dot_t(act, w2_ref[0])
    iota = jax.lax.broadcasted_iota(jnp.int32, we_ref.shape, 1)
    w_col = jnp.sum(jnp.where(iota == e, we_ref[...], 0.0),
                    axis=1, keepdims=True)
    out_ref[...] += w_col * eout * RM


def kernel(positions, hidden_states, residual, ln1_w, ln2_w, w_down, w_q_up,
           w_k_up, w_v_up, w_o, router_w, w1, w2):
    del positions, residual
    f32 = jnp.float32
    ln1 = ln1_w.reshape(1, H)
    ln2 = ln2_w.reshape(1, H)
    nbt = T // BT
    q, k, v = pl.pallas_call(
        _pre_attn_kernel,
        grid=(nbt,),
        in_specs=[
            pl.BlockSpec((BT, H), lambda i: (i, 0)),
            pl.BlockSpec((1, H), lambda i: (0, 0)),
            pl.BlockSpec((QC + 2 * KVC, H), lambda i: (0, 0)),
            pl.BlockSpec((H, QC), lambda i: (0, 0)),
            pl.BlockSpec((H, KVC), lambda i: (0, 0)),
            pl.BlockSpec((H, KVC), lambda i: (0, 0)),
        ],
        out_specs=[
            pl.BlockSpec((BT, H), lambda i: (i, 0)),
            pl.BlockSpec((BT, H), lambda i: (i, 0)),
            pl.BlockSpec((BT, H), lambda i: (i, 0)),
        ],
        out_shape=[jax.ShapeDtypeStruct((T, H), f32)] * 3,
    )(hidden_states, ln1, w_down, w_q_up, w_k_up, w_v_up)

    attn2d = pl.pallas_call(
        _flash_attn_kernel,
        grid=(NQB, NKB),
        in_specs=[
            pl.BlockSpec((BQ, H), lambda i, j: (i, 0)),
            pl.BlockSpec((BK, H), lambda i, j: (j, 0)),
            pl.BlockSpec((BK, H), lambda i, j: (j, 0)),
        ],
        out_specs=pl.BlockSpec((BQ, H), lambda i, j: (i, 0)),
        out_shape=jax.ShapeDtypeStruct((T, H), f32),
        scratch_shapes=[
            pltpu.VMEM((BQ, 128), f32),
            pltpu.VMEM((BQ, 128), f32),
            pltpu.VMEM((BQ, H), f32),
        ],
    )(q, k, v)

    res2, h2, we = pl.pallas_call(
        _post_attn_kernel,
        grid=(nbt,),
        in_specs=[
            pl.BlockSpec((BT, H), lambda i: (i, 0)),
            pl.BlockSpec((BT, H), lambda i: (i, 0)),
            pl.BlockSpec((1, H), lambda i: (0, 0)),
            pl.BlockSpec((H, H), lambda i: (0, 0)),
            pl.BlockSpec((E, H), lambda i: (0, 0)),
        ],
        out_specs=[
            pl.BlockSpec((BT, H), lambda i: (i, 0)),
            pl.BlockSpec((BT, H), lambda i: (i, 0)),
            pl.BlockSpec((BT, E), lambda i: (i, 0)),
        ],
        out_shape=[
            jax.ShapeDtypeStruct((T, H), f32),
            jax.ShapeDtypeStruct((T, H), f32),
            jax.ShapeDtypeStruct((T, E), f32),
        ],
    )(attn2d, hidden_states, ln2, w_o, router_w)

    out = pl.pallas_call(
        _moe_kernel,
        grid=(nbt, E),
        in_specs=[
            pl.BlockSpec((BT, H), lambda i, e: (i, 0)),
            pl.BlockSpec((BT, H), lambda i, e: (i, 0)),
            pl.BlockSpec((BT, E), lambda i, e: (i, 0)),
            pl.BlockSpec((1, 2 * FF, H), lambda i, e: (e, 0, 0)),
            pl.BlockSpec((1, H, FF), lambda i, e: (e, 0, 0)),
        ],
        out_specs=pl.BlockSpec((BT, H), lambda i, e: (i, 0)),
        out_shape=jax.ShapeDtypeStruct((T, H), f32),
    )(h2, res2, we, w1, w2)

    return (out, res2)
